# Initial kernel scaffold; baseline (speedup 1.0000x reference)
import jax
import jax.numpy as jnp
from jax.experimental import pallas as pl


def kernel(x_paper, emb_author, edge_cites, edge_writes, w_rel_cites_0, w_rel_writes_0, w_root_paper_0, b_root_paper_0, w_root_author_0, b_root_author_0, w_rel_cites_1, w_rel_writes_1, w_root_paper_1, b_root_paper_1, w_root_author_1, b_root_author_1, w_out, b_out):
    raise NotImplementedError("write your pallas kernel here")


# speedup vs baseline: 2.7815x; 2.7815x over previous
import functools
import jax
import jax.numpy as jnp
from jax import lax
from jax.experimental import pallas as pl
from jax.experimental.pallas import tpu as pltpu
from jax.experimental.pallas import tpu_sc as plsc

N = 10000
D = 128
E = 160000
NCLS = 349
NCLS_PAD = 384

NC = 2
NS = 16
K = 128
KC = 80
PAD = 3840
EPAD = E + PAD
EPT = EPAD // NS
ERAW = E // NS
NA = 10240
RPT = NA // NS
ZROW = 2 * N

_f32 = jnp.float32


def _sc_body(table, src, dst, zeros, ones, sums, cnts,
             acc, src_v, dst_v, rows_v, onesc_v, dstc_v, sem):
    c = lax.axis_index("c")
    s = lax.axis_index("s")
    r0 = s * RPT
    with_counts = cnts is not None
    pltpu.sync_copy(zeros.at[pl.ds(r0, RPT)], acc.at[pl.ds(r0, RPT)])
    if with_counts:
        pltpu.sync_copy(ones, onesc_v)
    plsc.subcore_barrier()

    base = c * EPAD + s * EPT

    def chunk(i, carry):
        off = base + i * K
        pltpu.sync_copy(src.at[pl.ds(off, K)], src_v)
        pltpu.sync_copy(dst.at[pl.ds(off, K)], dst_v)
        pltpu.async_copy(table.at[src_v], rows_v, sem).wait()
        pltpu.sync_copy(rows_v, acc.at[dst_v], add=True)
        return carry

    lax.fori_loop(0, EPT // K, chunk, 0)

    plsc.subcore_barrier()
    pltpu.sync_copy(acc.at[pl.ds(r0, RPT)], sums.at[c, pl.ds(r0, RPT)])

    if with_counts:
        plsc.subcore_barrier()
        pltpu.sync_copy(zeros.at[pl.ds(r0, RPT)], acc.at[pl.ds(r0, RPT)])
        plsc.subcore_barrier()
        cbase = c * EPAD + s * ERAW

        def cchunk(i, carry):
            off = cbase + i * KC
            pltpu.sync_copy(dst.at[pl.ds(off, KC)], dstc_v)
            pltpu.sync_copy(onesc_v, acc.at[dstc_v], add=True)
            return carry

        lax.fori_loop(0, ERAW // KC, cchunk, 0)
        plsc.subcore_barrier()
        pltpu.sync_copy(acc.at[pl.ds(r0, RPT)], cnts.at[c, pl.ds(r0, RPT)])


def _sc_body_counts(table, src, dst, zeros, ones, sums, cnts,
                    acc, src_v, dst_v, rows_v, onesc_v, dstc_v, sem):
    _sc_body(table, src, dst, zeros, ones, sums, cnts,
             acc, src_v, dst_v, rows_v, onesc_v, dstc_v, sem)


def _sc_body_sums(table, src, dst, zeros, sums,
                  acc, src_v, dst_v, rows_v, sem):
    _sc_body(table, src, dst, zeros, None, sums, None,
             acc, src_v, dst_v, rows_v, None, None, sem)


@functools.cache
def _sc_agg_counts():
    mesh = plsc.VectorSubcoreMesh(core_axis_name="c", subcore_axis_name="s",
                                  num_cores=NC, num_subcores=NS)
    return pl.kernel(
        _sc_body_counts,
        out_type=[jax.ShapeDtypeStruct((NC, NA, D), _f32),
                  jax.ShapeDtypeStruct((NC, NA, D), _f32)],
        mesh=mesh,
        scratch_types=[
            pltpu.VMEM_SHARED((NA, D), _f32),
            pltpu.VMEM((K,), jnp.int32),
            pltpu.VMEM((K,), jnp.int32),
            pltpu.VMEM((K, D), _f32),
            pltpu.VMEM((KC, D), _f32),
            pltpu.VMEM((KC,), jnp.int32),
            pltpu.SemaphoreType.DMA,
        ],
    )


@functools.cache
def _sc_agg():
    mesh = plsc.VectorSubcoreMesh(core_axis_name="c", subcore_axis_name="s",
                                  num_cores=NC, num_subcores=NS)
    return pl.kernel(
        _sc_body_sums,
        out_type=[jax.ShapeDtypeStruct((NC, NA, D), _f32)],
        mesh=mesh,
        scratch_types=[
            pltpu.VMEM_SHARED((NA, D), _f32),
            pltpu.VMEM((K,), jnp.int32),
            pltpu.VMEM((K,), jnp.int32),
            pltpu.VMEM((K, D), _f32),
            pltpu.SemaphoreType.DMA,
        ],
    )


BR = 1000


def _dot(a, b):
    return lax.dot_general(a, b, (((1,), (0,)), ((), ())),
                           preferred_element_type=_f32)


def _tc0_body(x_ref, emb_ref, sums_ref, cnts_ref, wrp_ref, brp_ref,
              wra_ref, bra_ref, rc_ref, rw_ref, h1_ref, a1_ref, h1t_ref):
    x = x_ref[...]
    cnt_c = jnp.maximum(cnts_ref[0, :, 0:1], 1.0)
    cnt_w = jnp.maximum(cnts_ref[1, :, 0:1], 1.0)
    agg_c = sums_ref[0] / cnt_c
    agg_w = sums_ref[1] / cnt_w
    wrp = wrp_ref[...]
    rc = rc_ref[...]
    rw = rw_ref[...]
    brp = brp_ref[...]
    out = _dot(x, wrp) + brp + _dot(agg_c, rc) + _dot(agg_w, rw)
    h1_ref[...] = jnp.maximum(out, 0.0)
    a1_ref[...] = jnp.maximum(_dot(emb_ref[...], wra_ref[...]) + bra_ref[...], 0.0)
    h1t_ref[...] = jnp.maximum(_dot(x, wrp + rc + rw) + brp, 0.0)


def _tc1_body(h1_ref, h1t_ref, sums_ref, cnts_ref, wrp_ref, brp_ref,
              rc_ref, rw_ref, wout_ref, bout_ref, logits_ref, alpha_ref):
    h1 = h1_ref[...]
    h1t = h1t_ref[...]
    cnt_c = jnp.maximum(cnts_ref[0, :, 0:1], 1.0)
    cnt_w = jnp.maximum(cnts_ref[1, :, 0:1], 1.0)
    agg_c = sums_ref[0] / cnt_c
    agg_w = sums_ref[1] / cnt_w
    wrp = wrp_ref[...]
    rc = rc_ref[...]
    rw = rw_ref[...]
    brp = brp_ref[...]
    h2 = jnp.maximum(_dot(h1, wrp) + brp + _dot(agg_c, rc) + _dot(agg_w, rw), 0.0)
    h2t = jnp.maximum(_dot(h1t, wrp + rc + rw) + brp, 0.0)
    num0 = jnp.sum(h1 * h1t, axis=-1, keepdims=True)
    den0 = (jnp.sqrt(jnp.sum(h1 * h1, axis=-1, keepdims=True))
            * jnp.sqrt(jnp.sum(h1t * h1t, axis=-1, keepdims=True)) + 1e-8)
    s0 = num0 / den0
    num1 = jnp.sum(h2 * h2t, axis=-1, keepdims=True)
    den1 = (jnp.sqrt(jnp.sum(h2 * h2, axis=-1, keepdims=True))
            * jnp.sqrt(jnp.sum(h2t * h2t, axis=-1, keepdims=True)) + 1e-8)
    s1 = num1 / den1
    m = jnp.maximum(s0, s1)
    e0 = jnp.exp(s0 - m)
    e1 = jnp.exp(s1 - m)
    tot = e0 + e1
    a0 = e0 / tot
    a1 = e1 / tot
    h = a0 * h1 + a1 * h2
    logits_ref[...] = _dot(h, wout_ref[...]) + bout_ref[...]
    lane = lax.broadcasted_iota(jnp.int32, (BR, D), 1)
    alpha_ref[...] = jnp.where(lane == 0, a0, jnp.where(lane == 1, a1, 0.0))


def _row_spec(shape):
    nd = len(shape)
    if nd == 2:
        return pl.BlockSpec((BR, shape[1]), lambda i: (i, 0))
    return pl.BlockSpec((shape[0], BR, shape[2]), lambda i: (0, i, 0))


def _full_spec(shape):
    nd = len(shape)
    return pl.BlockSpec(shape, lambda i: (0,) * nd)


def _tc_layer0(x, emb, sums, cnts, rc, rw, wrp, brp, wra, bra):
    grid = (N // BR,)
    return pl.pallas_call(
        _tc0_body,
        grid=grid,
        in_specs=[
            _row_spec((N, D)), _row_spec((N, D)),
            _row_spec((NC, N, D)), _row_spec((NC, N, D)),
            _full_spec((D, D)), _full_spec((1, D)),
            _full_spec((D, D)), _full_spec((1, D)),
            _full_spec((D, D)), _full_spec((D, D)),
        ],
        out_specs=[_row_spec((N, D)), _row_spec((N, D)), _row_spec((N, D))],
        out_shape=[jax.ShapeDtypeStruct((N, D), _f32)] * 3,
    )(x, emb, sums, cnts, wrp, brp.reshape(1, D), wra, bra.reshape(1, D), rc, rw)


def _tc_layer1(h1, h1t, sums, cnts, rc, rw, wrp, brp, wout_p, bout_p):
    grid = (N // BR,)
    return pl.pallas_call(
        _tc1_body,
        grid=grid,
        in_specs=[
            _row_spec((N, D)), _row_spec((N, D)),
            _row_spec((NC, N, D)), _row_spec((NC, N, D)),
            _full_spec((D, D)), _full_spec((1, D)),
            _full_spec((D, D)), _full_spec((D, D)),
            _full_spec((D, NCLS_PAD)), _full_spec((1, NCLS_PAD)),
        ],
        out_specs=[_row_spec((N, NCLS_PAD)), _row_spec((N, D))],
        out_shape=[jax.ShapeDtypeStruct((N, NCLS_PAD), _f32),
                   jax.ShapeDtypeStruct((N, D), _f32)],
    )(h1, h1t, sums, cnts, wrp, brp.reshape(1, D), rc, rw, wout_p, bout_p)


def kernel(x_paper, emb_author, edge_cites, edge_writes,
           w_rel_cites_0, w_rel_writes_0, w_root_paper_0, b_root_paper_0,
           w_root_author_0, b_root_author_0,
           w_rel_cites_1, w_rel_writes_1, w_root_paper_1, b_root_paper_1,
           w_root_author_1, b_root_author_1, w_out, b_out):
    i32 = jnp.int32
    zrow8 = jnp.zeros((8, D), _f32)
    table0 = jnp.concatenate([x_paper, emb_author, zrow8], axis=0)
    pad_src = jnp.full((PAD,), ZROW, i32)
    pad_dst = jnp.zeros((PAD,), i32)
    src_all = jnp.concatenate(
        [edge_cites[0], pad_src, edge_writes[0] + N, pad_src])
    dst_all = jnp.concatenate(
        [edge_cites[1], pad_dst, edge_writes[1], pad_dst])
    zeros128 = jnp.zeros((NA, D), _f32)
    ones = jnp.ones((KC, D), _f32)

    sums0, cnts = _sc_agg_counts()(table0, src_all, dst_all, zeros128, ones)
    h1, a1, h1t = _tc_layer0(x_paper, emb_author, sums0, cnts,
                             w_rel_cites_0, w_rel_writes_0,
                             w_root_paper_0, b_root_paper_0,
                             w_root_author_0, b_root_author_0)
    table1 = jnp.concatenate([h1, a1, zrow8], axis=0)
    (sums1,) = _sc_agg()(table1, src_all, dst_all, zeros128)
    wout_p = jnp.concatenate(
        [w_out, jnp.zeros((D, NCLS_PAD - NCLS), _f32)], axis=1)
    bout_p = jnp.concatenate(
        [b_out, jnp.zeros((NCLS_PAD - NCLS,), _f32)]).reshape(1, NCLS_PAD)
    logits_p, alpha_p = _tc_layer1(h1, h1t, sums1, cnts,
                                   w_rel_cites_1, w_rel_writes_1,
                                   w_root_paper_1, b_root_paper_1,
                                   wout_p, bout_p)
    return logits_p[:, :NCLS], alpha_p[:, :2]


# speedup vs baseline: 3.2829x; 1.1803x over previous
import functools
import jax
import jax.numpy as jnp
from jax import lax
from jax.experimental import pallas as pl
from jax.experimental.pallas import tpu as pltpu
from jax.experimental.pallas import tpu_sc as plsc

N = 10000
D = 128
E = 160000
NCLS = 349
NCLS_PAD = 384

NC = 2
NS = 16
K = 128
PAD = 3840
EPAD = E + PAD
EPT = EPAD // NS
ERAW = E // NS
NA = 10240
RPT = NA // NS

_f32 = jnp.float32


NCH = EPT // K
NST = NCH // 2


def _gstart(table, src, dst, off, src_v, dst_v, rows_v, sem):
    pltpu.sync_copy(src.at[pl.ds(off, K)], src_v)
    pltpu.sync_copy(dst.at[pl.ds(off, K)], dst_v)
    pltpu.async_copy(table.at[src_v], rows_v, sem)


def _gwait(table, src_v, rows_v, sem):
    pltpu.make_async_copy(table.at[src_v], rows_v, sem).wait()


def _sc_body(table, src, dst, zeros, ones, sums, cnts, acc,
             src_v0, dst_v0, rows_v0, src_v1, dst_v1, rows_v1,
             sem0, sem1):
    c = lax.axis_index("c")
    s = lax.axis_index("s")
    r0 = s * RPT
    with_counts = cnts is not None
    pltpu.sync_copy(zeros.at[pl.ds(r0, RPT)], acc.at[pl.ds(r0, RPT)])
    plsc.subcore_barrier()

    base = c * EPAD + s * EPT

    _gstart(table, src, dst, base, src_v0, dst_v0, rows_v0, sem0)

    def step(i, carry):
        _gstart(table, src, dst, base + (2 * i + 1) * K,
                src_v1, dst_v1, rows_v1, sem1)
        _gwait(table, src_v0, rows_v0, sem0)
        pltpu.sync_copy(rows_v0, acc.at[dst_v0], add=True)

        @pl.when(i < NST - 1)
        def _():
            _gstart(table, src, dst, base + (2 * i + 2) * K,
                    src_v0, dst_v0, rows_v0, sem0)

        _gwait(table, src_v1, rows_v1, sem1)
        pltpu.sync_copy(rows_v1, acc.at[dst_v1], add=True)
        return carry

    lax.fori_loop(0, NST, step, 0)

    plsc.subcore_barrier()
    pltpu.sync_copy(acc.at[pl.ds(r0, RPT)], sums.at[c, pl.ds(r0, RPT)])

    if with_counts:
        plsc.subcore_barrier()
        pltpu.sync_copy(zeros.at[pl.ds(r0, RPT)], acc.at[pl.ds(r0, RPT)])
        pltpu.sync_copy(ones, rows_v0)
        plsc.subcore_barrier()

        pltpu.sync_copy(dst.at[pl.ds(base, K)], dst_v0)
        pltpu.async_copy(rows_v0, acc.at[dst_v0], sem0, add=True)

        def cstep(i, carry):
            pltpu.sync_copy(dst.at[pl.ds(base + (2 * i + 1) * K, K)], dst_v1)
            pltpu.async_copy(rows_v0, acc.at[dst_v1], sem1, add=True)
            pltpu.make_async_copy(rows_v0, acc.at[dst_v0], sem0).wait()

            @pl.when(i < NST - 1)
            def _():
                pltpu.sync_copy(dst.at[pl.ds(base + (2 * i + 2) * K, K)], dst_v0)
                pltpu.async_copy(rows_v0, acc.at[dst_v0], sem0, add=True)

            pltpu.make_async_copy(rows_v0, acc.at[dst_v1], sem1).wait()
            return carry

        lax.fori_loop(0, NST, cstep, 0)
        plsc.subcore_barrier()
        pltpu.sync_copy(acc.at[pl.ds(r0, RPT)], cnts.at[c, pl.ds(r0, RPT)])


def _sc_body_counts(table, src, dst, zeros, ones, sums, cnts, acc,
                    src_v0, dst_v0, rows_v0, src_v1, dst_v1, rows_v1,
                    sem0, sem1):
    _sc_body(table, src, dst, zeros, ones, sums, cnts, acc,
             src_v0, dst_v0, rows_v0, src_v1, dst_v1, rows_v1,
             sem0, sem1)


def _sc_body_sums(table, src, dst, zeros, sums, acc,
                  src_v0, dst_v0, rows_v0, src_v1, dst_v1, rows_v1,
                  sem0, sem1):
    _sc_body(table, src, dst, zeros, None, sums, None, acc,
             src_v0, dst_v0, rows_v0, src_v1, dst_v1, rows_v1,
             sem0, sem1)


@functools.cache
def _sc_agg_counts():
    mesh = plsc.VectorSubcoreMesh(core_axis_name="c", subcore_axis_name="s",
                                  num_cores=NC, num_subcores=NS)
    return pl.kernel(
        _sc_body_counts,
        out_type=[jax.ShapeDtypeStruct((NC, NA, D), _f32),
                  jax.ShapeDtypeStruct((NC, NA, D), _f32)],
        mesh=mesh,
        scratch_types=[
            pltpu.VMEM_SHARED((NA, D), _f32),
            pltpu.VMEM((K,), jnp.int32),
            pltpu.VMEM((K,), jnp.int32),
            pltpu.VMEM((K, D), _f32),
            pltpu.VMEM((K,), jnp.int32),
            pltpu.VMEM((K,), jnp.int32),
            pltpu.VMEM((K, D), _f32),
            pltpu.SemaphoreType.DMA,
            pltpu.SemaphoreType.DMA,
        ],
    )


@functools.cache
def _sc_agg():
    mesh = plsc.VectorSubcoreMesh(core_axis_name="c", subcore_axis_name="s",
                                  num_cores=NC, num_subcores=NS)
    return pl.kernel(
        _sc_body_sums,
        out_type=[jax.ShapeDtypeStruct((NC, NA, D), _f32)],
        mesh=mesh,
        scratch_types=[
            pltpu.VMEM_SHARED((NA, D), _f32),
            pltpu.VMEM((K,), jnp.int32),
            pltpu.VMEM((K,), jnp.int32),
            pltpu.VMEM((K, D), _f32),
            pltpu.VMEM((K,), jnp.int32),
            pltpu.VMEM((K,), jnp.int32),
            pltpu.VMEM((K, D), _f32),
            pltpu.SemaphoreType.DMA,
            pltpu.SemaphoreType.DMA,
        ],
    )


BR = 1000


def _dot(a, b):
    return lax.dot_general(a, b, (((1,), (0,)), ((), ())),
                           preferred_element_type=_f32)


def _tc0_body(x_ref, emb_ref, sums_ref, cnts_ref, wrp_ref, brp_ref,
              wra_ref, bra_ref, rc_ref, rw_ref, h1_ref, a1_ref, h1t_ref):
    x = x_ref[...]
    cnt_c = jnp.maximum(cnts_ref[0, :, 0:1], 1.0)
    cnt_w = jnp.maximum(cnts_ref[1, :, 0:1], 1.0)
    agg_c = sums_ref[0] / cnt_c
    agg_w = sums_ref[1] / cnt_w
    wrp = wrp_ref[...]
    rc = rc_ref[...]
    rw = rw_ref[...]
    brp = brp_ref[...]
    out = _dot(x, wrp) + brp + _dot(agg_c, rc) + _dot(agg_w, rw)
    h1_ref[...] = jnp.maximum(out, 0.0)
    a1_ref[...] = jnp.maximum(_dot(emb_ref[...], wra_ref[...]) + bra_ref[...], 0.0)
    h1t_ref[...] = jnp.maximum(_dot(x, wrp + rc + rw) + brp, 0.0)


def _tc1_body(h1_ref, h1t_ref, sums_ref, cnts_ref, wrp_ref, brp_ref,
              rc_ref, rw_ref, wout_ref, bout_ref, logits_ref, alpha_ref):
    h1 = h1_ref[...]
    h1t = h1t_ref[...]
    cnt_c = jnp.maximum(cnts_ref[0, :, 0:1], 1.0)
    cnt_w = jnp.maximum(cnts_ref[1, :, 0:1], 1.0)
    agg_c = sums_ref[0] / cnt_c
    agg_w = sums_ref[1] / cnt_w
    wrp = wrp_ref[...]
    rc = rc_ref[...]
    rw = rw_ref[...]
    brp = brp_ref[...]
    h2 = jnp.maximum(_dot(h1, wrp) + brp + _dot(agg_c, rc) + _dot(agg_w, rw), 0.0)
    h2t = jnp.maximum(_dot(h1t, wrp + rc + rw) + brp, 0.0)
    num0 = jnp.sum(h1 * h1t, axis=-1, keepdims=True)
    den0 = (jnp.sqrt(jnp.sum(h1 * h1, axis=-1, keepdims=True))
            * jnp.sqrt(jnp.sum(h1t * h1t, axis=-1, keepdims=True)) + 1e-8)
    s0 = num0 / den0
    num1 = jnp.sum(h2 * h2t, axis=-1, keepdims=True)
    den1 = (jnp.sqrt(jnp.sum(h2 * h2, axis=-1, keepdims=True))
            * jnp.sqrt(jnp.sum(h2t * h2t, axis=-1, keepdims=True)) + 1e-8)
    s1 = num1 / den1
    m = jnp.maximum(s0, s1)
    e0 = jnp.exp(s0 - m)
    e1 = jnp.exp(s1 - m)
    tot = e0 + e1
    a0 = e0 / tot
    a1 = e1 / tot
    h = a0 * h1 + a1 * h2
    logits_ref[...] = _dot(h, wout_ref[...]) + bout_ref[...]
    lane = lax.broadcasted_iota(jnp.int32, (BR, D), 1)
    alpha_ref[...] = jnp.where(lane == 0, a0, jnp.where(lane == 1, a1, 0.0))


def _row_spec(shape):
    nd = len(shape)
    if nd == 2:
        return pl.BlockSpec((BR, shape[1]), lambda i: (i, 0))
    return pl.BlockSpec((shape[0], BR, shape[2]), lambda i: (0, i, 0))


def _full_spec(shape):
    nd = len(shape)
    return pl.BlockSpec(shape, lambda i: (0,) * nd)


def _tc_layer0(x, emb, sums, cnts, rc, rw, wrp, brp, wra, bra):
    grid = (N // BR,)
    return pl.pallas_call(
        _tc0_body,
        grid=grid,
        in_specs=[
            _row_spec((N, D)), _row_spec((N, D)),
            _row_spec((NC, N, D)), _row_spec((NC, N, D)),
            _full_spec((D, D)), _full_spec((1, D)),
            _full_spec((D, D)), _full_spec((1, D)),
            _full_spec((D, D)), _full_spec((D, D)),
        ],
        out_specs=[_row_spec((N, D)), _row_spec((N, D)), _row_spec((N, D))],
        out_shape=[jax.ShapeDtypeStruct((N, D), _f32)] * 3,
    )(x, emb, sums, cnts, wrp, brp.reshape(1, D), wra, bra.reshape(1, D), rc, rw)


def _tc_layer1(h1, h1t, sums, cnts, rc, rw, wrp, brp, wout_p, bout_p):
    grid = (N // BR,)
    return pl.pallas_call(
        _tc1_body,
        grid=grid,
        in_specs=[
            _row_spec((N, D)), _row_spec((N, D)),
            _row_spec((NC, N, D)), _row_spec((NC, N, D)),
            _full_spec((D, D)), _full_spec((1, D)),
            _full_spec((D, D)), _full_spec((D, D)),
            _full_spec((D, NCLS_PAD)), _full_spec((1, NCLS_PAD)),
        ],
        out_specs=[_row_spec((N, NCLS_PAD)), _row_spec((N, D))],
        out_shape=[jax.ShapeDtypeStruct((N, NCLS_PAD), _f32),
                   jax.ShapeDtypeStruct((N, D), _f32)],
    )(h1, h1t, sums, cnts, wrp, brp.reshape(1, D), rc, rw, wout_p, bout_p)


def kernel(x_paper, emb_author, edge_cites, edge_writes,
           w_rel_cites_0, w_rel_writes_0, w_root_paper_0, b_root_paper_0,
           w_root_author_0, b_root_author_0,
           w_rel_cites_1, w_rel_writes_1, w_root_paper_1, b_root_paper_1,
           w_root_author_1, b_root_author_1, w_out, b_out):
    i32 = jnp.int32
    table0 = jnp.concatenate([x_paper, emb_author], axis=0)
    pad_src = jnp.zeros((PAD,), i32)
    pad_dst = jnp.full((PAD,), N, i32)
    src_all = jnp.concatenate(
        [edge_cites[0], pad_src, edge_writes[0] + N, pad_src])
    dst_all = jnp.concatenate(
        [edge_cites[1], pad_dst, edge_writes[1], pad_dst])
    zeros128 = jnp.zeros((NA, D), _f32)
    ones = jnp.ones((K, D), _f32)

    sums0, cnts = _sc_agg_counts()(table0, src_all, dst_all, zeros128, ones)
    h1, a1, h1t = _tc_layer0(x_paper, emb_author, sums0, cnts,
                             w_rel_cites_0, w_rel_writes_0,
                             w_root_paper_0, b_root_paper_0,
                             w_root_author_0, b_root_author_0)
    table1 = jnp.concatenate([h1, a1], axis=0)
    (sums1,) = _sc_agg()(table1, src_all, dst_all, zeros128)
    wout_p = jnp.concatenate(
        [w_out, jnp.zeros((D, NCLS_PAD - NCLS), _f32)], axis=1)
    bout_p = jnp.concatenate(
        [b_out, jnp.zeros((NCLS_PAD - NCLS,), _f32)]).reshape(1, NCLS_PAD)
    logits_p, alpha_p = _tc_layer1(h1, h1t, sums1, cnts,
                                   w_rel_cites_1, w_rel_writes_1,
                                   w_root_paper_1, b_root_paper_1,
                                   wout_p, bout_p)
    return logits_p[:, :NCLS], alpha_p[:, :2]


# speedup vs baseline: 4.2227x; 1.2863x over previous
import functools
import jax
import jax.numpy as jnp
from jax import lax
from jax.experimental import pallas as pl
from jax.experimental.pallas import tpu as pltpu
from jax.experimental.pallas import tpu_sc as plsc

N = 10000
D = 128
E = 160000
NCLS = 349
NCLS_PAD = 384

NC = 2
NS = 16
K = 128
PAD = 3840
EPAD = E + PAD
EPT = EPAD // NS
NA = 10240
RPT = NA // NS
NCH = EPT // K
NST4 = NCH // 4
CPT = NCH
CPC = EPAD // K

_f32 = jnp.float32


def _sum_pass(table, slab, acc, base_cid, ibufs, isems, rows, gsems):
    def istart(q, b):
        pltpu.async_copy(slab.at[base_cid + q], ibufs[b], isems[b])

    def iwait(b):
        pltpu.make_async_copy(slab.at[0], ibufs[b], isems[b]).wait()

    def gstart(b, rb):
        iwait(b)
        pltpu.async_copy(table.at[ibufs[b].at[0]], rows[rb], gsems[rb])

    def gwait(rb):
        pltpu.make_async_copy(table.at[ibufs[0].at[0]], rows[rb],
                              gsems[rb]).wait()

    for b in range(4):
        istart(b, b)
    gstart(0, 0)

    def step(i, carry):
        for t in range(4):
            rb = t % 2
            nrb = (t + 1) % 2
            if t < 3:
                gstart(t + 1, nrb)
            else:
                @pl.when(i < NST4 - 1)
                def _():
                    gstart(0, nrb)
            gwait(rb)
            pltpu.sync_copy(rows[rb], acc.at[ibufs[t].at[1]], add=True)

            @pl.when(i < NST4 - 1)
            def _():
                istart(4 * i + 4 + t, t)
        return carry

    lax.fori_loop(0, NST4, step, 0)


def _count_pass(slab, acc, base_cid, ibufs, isems, ones_v):
    def istart(q, b):
        pltpu.async_copy(slab.at[base_cid + q], ibufs[b], isems[b])

    def iwait(b):
        pltpu.make_async_copy(slab.at[0], ibufs[b], isems[b]).wait()

    for b in range(4):
        istart(b, b)

    def step(i, carry):
        for t in range(4):
            iwait(t)
            pltpu.sync_copy(ones_v, acc.at[ibufs[t].at[1]], add=True)

            @pl.when(i < NST4 - 1)
            def _():
                istart(4 * i + 4 + t, t)
        return carry

    lax.fori_loop(0, NST4, step, 0)


def _sc_body(table_p, table_a, slab, zeros, ones, sums, cnts, acc,
             ibuf0, ibuf1, ibuf2, ibuf3, rows0, rows1,
             isem0, isem1, isem2, isem3, gsem0, gsem1):
    c = lax.axis_index("c")
    s = lax.axis_index("s")
    r0 = s * RPT
    ibufs = (ibuf0, ibuf1, ibuf2, ibuf3)
    isems = (isem0, isem1, isem2, isem3)
    rows = (rows0, rows1)
    gsems = (gsem0, gsem1)
    with_counts = cnts is not None

    pltpu.sync_copy(zeros.at[pl.ds(r0, RPT)], acc.at[pl.ds(r0, RPT)])
    plsc.subcore_barrier()

    base_cid = c * CPC + s * CPT

    @pl.when(c == 0)
    def _():
        _sum_pass(table_p, slab, acc, base_cid, ibufs, isems, rows, gsems)

    @pl.when(c == 1)
    def _():
        _sum_pass(table_a, slab, acc, base_cid, ibufs, isems, rows, gsems)

    plsc.subcore_barrier()
    pltpu.sync_copy(acc.at[pl.ds(r0, RPT)], sums.at[c, pl.ds(r0, RPT)])

    if with_counts:
        plsc.subcore_barrier()
        pltpu.sync_copy(zeros.at[pl.ds(r0, RPT)], acc.at[pl.ds(r0, RPT)])
        pltpu.sync_copy(ones, rows0)
        plsc.subcore_barrier()
        _count_pass(slab, acc, base_cid, ibufs, isems, rows0)
        plsc.subcore_barrier()
        pltpu.sync_copy(acc.at[pl.ds(r0, RPT)], cnts.at[c, pl.ds(r0, RPT)])


def _sc_body_counts(table_p, table_a, slab, zeros, ones, sums, cnts, acc,
                    ibuf0, ibuf1, ibuf2, ibuf3, rows0, rows1,
                    isem0, isem1, isem2, isem3, gsem0, gsem1):
    _sc_body(table_p, table_a, slab, zeros, ones, sums, cnts, acc,
             ibuf0, ibuf1, ibuf2, ibuf3, rows0, rows1,
             isem0, isem1, isem2, isem3, gsem0, gsem1)


def _sc_body_sums(table_p, table_a, slab, zeros, sums, acc,
                  ibuf0, ibuf1, ibuf2, ibuf3, rows0, rows1,
                  isem0, isem1, isem2, isem3, gsem0, gsem1):
    _sc_body(table_p, table_a, slab, zeros, None, sums, None, acc,
             ibuf0, ibuf1, ibuf2, ibuf3, rows0, rows1,
             isem0, isem1, isem2, isem3, gsem0, gsem1)


def _sc_scratch():
    return [
        pltpu.VMEM_SHARED((NA, D), _f32),
        pltpu.VMEM((2, K), jnp.int32),
        pltpu.VMEM((2, K), jnp.int32),
        pltpu.VMEM((2, K), jnp.int32),
        pltpu.VMEM((2, K), jnp.int32),
        pltpu.VMEM((K, D), _f32),
        pltpu.VMEM((K, D), _f32),
        pltpu.SemaphoreType.DMA,
        pltpu.SemaphoreType.DMA,
        pltpu.SemaphoreType.DMA,
        pltpu.SemaphoreType.DMA,
        pltpu.SemaphoreType.DMA,
        pltpu.SemaphoreType.DMA,
    ]


@functools.cache
def _sc_agg_counts():
    mesh = plsc.VectorSubcoreMesh(core_axis_name="c", subcore_axis_name="s",
                                  num_cores=NC, num_subcores=NS)
    return pl.kernel(
        _sc_body_counts,
        out_type=[jax.ShapeDtypeStruct((NC, NA, D), _f32),
                  jax.ShapeDtypeStruct((NC, NA, D), _f32)],
        mesh=mesh,
        scratch_types=_sc_scratch(),
    )


@functools.cache
def _sc_agg():
    mesh = plsc.VectorSubcoreMesh(core_axis_name="c", subcore_axis_name="s",
                                  num_cores=NC, num_subcores=NS)
    return pl.kernel(
        _sc_body_sums,
        out_type=[jax.ShapeDtypeStruct((NC, NA, D), _f32)],
        mesh=mesh,
        scratch_types=_sc_scratch(),
    )


BR = 1000


def _dot(a, b):
    return lax.dot_general(a, b, (((1,), (0,)), ((), ())),
                           preferred_element_type=_f32)


def _tc0_body(x_ref, emb_ref, sums_ref, cnts_ref, wrp_ref, brp_ref,
              wra_ref, bra_ref, rc_ref, rw_ref, h1_ref, a1_ref, h1t_ref):
    x = x_ref[...]
    cnt_c = jnp.maximum(cnts_ref[0, :, 0:1], 1.0)
    cnt_w = jnp.maximum(cnts_ref[1, :, 0:1], 1.0)
    agg_c = sums_ref[0] / cnt_c
    agg_w = sums_ref[1] / cnt_w
    wrp = wrp_ref[...]
    rc = rc_ref[...]
    rw = rw_ref[...]
    brp = brp_ref[...]
    out = _dot(x, wrp) + brp + _dot(agg_c, rc) + _dot(agg_w, rw)
    h1_ref[...] = jnp.maximum(out, 0.0)
    a1_ref[...] = jnp.maximum(_dot(emb_ref[...], wra_ref[...]) + bra_ref[...], 0.0)
    h1t_ref[...] = jnp.maximum(_dot(x, wrp + rc + rw) + brp, 0.0)


def _tc1_body(h1_ref, h1t_ref, sums_ref, cnts_ref, wrp_ref, brp_ref,
              rc_ref, rw_ref, wout_ref, bout_ref, logits_ref, alpha_ref):
    h1 = h1_ref[...]
    h1t = h1t_ref[...]
    cnt_c = jnp.maximum(cnts_ref[0, :, 0:1], 1.0)
    cnt_w = jnp.maximum(cnts_ref[1, :, 0:1], 1.0)
    agg_c = sums_ref[0] / cnt_c
    agg_w = sums_ref[1] / cnt_w
    wrp = wrp_ref[...]
    rc = rc_ref[...]
    rw = rw_ref[...]
    brp = brp_ref[...]
    h2 = jnp.maximum(_dot(h1, wrp) + brp + _dot(agg_c, rc) + _dot(agg_w, rw), 0.0)
    h2t = jnp.maximum(_dot(h1t, wrp + rc + rw) + brp, 0.0)
    num0 = jnp.sum(h1 * h1t, axis=-1, keepdims=True)
    den0 = (jnp.sqrt(jnp.sum(h1 * h1, axis=-1, keepdims=True))
            * jnp.sqrt(jnp.sum(h1t * h1t, axis=-1, keepdims=True)) + 1e-8)
    s0 = num0 / den0
    num1 = jnp.sum(h2 * h2t, axis=-1, keepdims=True)
    den1 = (jnp.sqrt(jnp.sum(h2 * h2, axis=-1, keepdims=True))
            * jnp.sqrt(jnp.sum(h2t * h2t, axis=-1, keepdims=True)) + 1e-8)
    s1 = num1 / den1
    m = jnp.maximum(s0, s1)
    e0 = jnp.exp(s0 - m)
    e1 = jnp.exp(s1 - m)
    tot = e0 + e1
    a0 = e0 / tot
    a1 = e1 / tot
    h = a0 * h1 + a1 * h2
    logits_ref[...] = _dot(h, wout_ref[...]) + bout_ref[...]
    lane = lax.broadcasted_iota(jnp.int32, (BR, D), 1)
    alpha_ref[...] = jnp.where(lane == 0, a0, jnp.where(lane == 1, a1, 0.0))


def _row_spec(shape):
    nd = len(shape)
    if nd == 2:
        return pl.BlockSpec((BR, shape[1]), lambda i: (i, 0))
    return pl.BlockSpec((shape[0], BR, shape[2]), lambda i: (0, i, 0))


def _full_spec(shape):
    nd = len(shape)
    return pl.BlockSpec(shape, lambda i: (0,) * nd)


def _tc_layer0(x, emb, sums, cnts, rc, rw, wrp, brp, wra, bra):
    grid = (N // BR,)
    return pl.pallas_call(
        _tc0_body,
        grid=grid,
        in_specs=[
            _row_spec((N, D)), _row_spec((N, D)),
            _row_spec((NC, N, D)), _row_spec((NC, N, D)),
            _full_spec((D, D)), _full_spec((1, D)),
            _full_spec((D, D)), _full_spec((1, D)),
            _full_spec((D, D)), _full_spec((D, D)),
        ],
        out_specs=[_row_spec((N, D)), _row_spec((N, D)), _row_spec((N, D))],
        out_shape=[jax.ShapeDtypeStruct((N, D), _f32)] * 3,
    )(x, emb, sums, cnts, wrp, brp.reshape(1, D), wra, bra.reshape(1, D), rc, rw)


def _tc_layer1(h1, h1t, sums, cnts, rc, rw, wrp, brp, wout_p, bout_p):
    grid = (N // BR,)
    return pl.pallas_call(
        _tc1_body,
        grid=grid,
        in_specs=[
            _row_spec((N, D)), _row_spec((N, D)),
            _row_spec((NC, N, D)), _row_spec((NC, N, D)),
            _full_spec((D, D)), _full_spec((1, D)),
            _full_spec((D, D)), _full_spec((D, D)),
            _full_spec((D, NCLS_PAD)), _full_spec((1, NCLS_PAD)),
        ],
        out_specs=[_row_spec((N, NCLS_PAD)), _row_spec((N, D))],
        out_shape=[jax.ShapeDtypeStruct((N, NCLS_PAD), _f32),
                   jax.ShapeDtypeStruct((N, D), _f32)],
    )(h1, h1t, sums, cnts, wrp, brp.reshape(1, D), rc, rw, wout_p, bout_p)


def kernel(x_paper, emb_author, edge_cites, edge_writes,
           w_rel_cites_0, w_rel_writes_0, w_root_paper_0, b_root_paper_0,
           w_root_author_0, b_root_author_0,
           w_rel_cites_1, w_rel_writes_1, w_root_paper_1, b_root_paper_1,
           w_root_author_1, b_root_author_1, w_out, b_out):
    i32 = jnp.int32
    pad_src = jnp.zeros((PAD,), i32)
    pad_dst = jnp.full((PAD,), N, i32)
    src_all = jnp.concatenate(
        [edge_cites[0], pad_src, edge_writes[0], pad_src])
    dst_all = jnp.concatenate(
        [edge_cites[1], pad_dst, edge_writes[1], pad_dst])
    slab = jnp.stack([src_all.reshape(-1, K), dst_all.reshape(-1, K)], axis=1)
    zeros128 = jnp.zeros((NA, D), _f32)
    ones = jnp.ones((K, D), _f32)

    sums0, cnts = _sc_agg_counts()(x_paper, emb_author, slab, zeros128, ones)
    h1, a1, h1t = _tc_layer0(x_paper, emb_author, sums0, cnts,
                             w_rel_cites_0, w_rel_writes_0,
                             w_root_paper_0, b_root_paper_0,
                             w_root_author_0, b_root_author_0)
    (sums1,) = _sc_agg()(h1, a1, slab, zeros128)
    wout_p = jnp.concatenate(
        [w_out, jnp.zeros((D, NCLS_PAD - NCLS), _f32)], axis=1)
    bout_p = jnp.concatenate(
        [b_out, jnp.zeros((NCLS_PAD - NCLS,), _f32)]).reshape(1, NCLS_PAD)
    logits_p, alpha_p = _tc_layer1(h1, h1t, sums1, cnts,
                                   w_rel_cites_1, w_rel_writes_1,
                                   w_root_paper_1, b_root_paper_1,
                                   wout_p, bout_p)
    return logits_p[:, :NCLS], alpha_p[:, :2]


# speedup vs baseline: 4.6914x; 1.1110x over previous
import functools
import jax
import jax.numpy as jnp
from jax import lax
from jax.experimental import pallas as pl
from jax.experimental.pallas import tpu as pltpu
from jax.experimental.pallas import tpu_sc as plsc

N = 10000
D = 128
E = 160000
NCLS = 349
NCLS_PAD = 384

NC = 2
NS = 16
K = 128
PAD = 3840
EPAD = E + PAD
EPT = EPAD // NS
NA = 10240
RPT = NA // NS
NCH = EPT // K
NST4 = NCH // 4
CPT = NCH
CPC = EPAD // K

_f32 = jnp.float32


def _sum_pass(table, src2d, dst2d, acc, base_cid,
              sbufs, dbufs, isems, rows, gsems):
    def istart(q, b):
        pltpu.async_copy(src2d.at[base_cid + q], sbufs[b], isems[b])
        pltpu.async_copy(dst2d.at[base_cid + q], dbufs[b], isems[b])

    def iwait(b):
        pltpu.make_async_copy(src2d.at[0], sbufs[b], isems[b]).wait()
        pltpu.make_async_copy(dst2d.at[0], dbufs[b], isems[b]).wait()

    def gstart(b, rb):
        iwait(b)
        pltpu.async_copy(table.at[sbufs[b]], rows[rb], gsems[rb])

    def gwait(rb):
        pltpu.make_async_copy(table.at[sbufs[0]], rows[rb],
                              gsems[rb]).wait()

    for b in range(4):
        istart(b, b)
    gstart(0, 0)

    def step(i, carry):
        for t in range(4):
            rb = t % 2
            nrb = (t + 1) % 2
            if t < 3:
                gstart(t + 1, nrb)
            else:
                @pl.when(i < NST4 - 1)
                def _():
                    gstart(0, nrb)
            gwait(rb)
            pltpu.sync_copy(rows[rb], acc.at[dbufs[t]], add=True)

            @pl.when(i < NST4 - 1)
            def _():
                istart(4 * i + 4 + t, t)
        return carry

    lax.fori_loop(0, NST4, step, 0)


def _count_pass(dst2d, acc, base_cid, dbufs, isems, ones_v):
    def istart(q, b):
        pltpu.async_copy(dst2d.at[base_cid + q], dbufs[b], isems[b])

    def iwait(b):
        pltpu.make_async_copy(dst2d.at[0], dbufs[b], isems[b]).wait()

    for b in range(4):
        istart(b, b)

    def step(i, carry):
        for t in range(4):
            iwait(t)
            pltpu.sync_copy(ones_v, acc.at[dbufs[t]], add=True)

            @pl.when(i < NST4 - 1)
            def _():
                istart(4 * i + 4 + t, t)
        return carry

    lax.fori_loop(0, NST4, step, 0)


def _sc_body(table_p, table_a, src2d, dst2d, zeros, ones, sums, cnts, acc,
             sbuf0, sbuf1, sbuf2, sbuf3, dbuf0, dbuf1, dbuf2, dbuf3,
             rows0, rows1,
             isem0, isem1, isem2, isem3, gsem0, gsem1):
    c = lax.axis_index("c")
    s = lax.axis_index("s")
    r0 = s * RPT
    sbufs = (sbuf0, sbuf1, sbuf2, sbuf3)
    dbufs = (dbuf0, dbuf1, dbuf2, dbuf3)
    isems = (isem0, isem1, isem2, isem3)
    rows = (rows0, rows1)
    gsems = (gsem0, gsem1)
    with_counts = cnts is not None

    pltpu.sync_copy(zeros.at[pl.ds(r0, RPT)], acc.at[pl.ds(r0, RPT)])
    plsc.subcore_barrier()

    base_cid = c * CPC + s * CPT

    @pl.when(c == 0)
    def _():
        _sum_pass(table_p, src2d, dst2d, acc, base_cid,
                  sbufs, dbufs, isems, rows, gsems)

    @pl.when(c == 1)
    def _():
        _sum_pass(table_a, src2d, dst2d, acc, base_cid,
                  sbufs, dbufs, isems, rows, gsems)

    plsc.subcore_barrier()
    pltpu.sync_copy(acc.at[pl.ds(r0, RPT)], sums.at[c, pl.ds(r0, RPT)])

    if with_counts:
        plsc.subcore_barrier()
        pltpu.sync_copy(zeros.at[pl.ds(r0, RPT)], acc.at[pl.ds(r0, RPT)])
        pltpu.sync_copy(ones, rows0)
        plsc.subcore_barrier()
        _count_pass(dst2d, acc, base_cid, dbufs, isems, rows0)
        plsc.subcore_barrier()
        pltpu.sync_copy(acc.at[pl.ds(r0, RPT)], cnts.at[c, pl.ds(r0, RPT)])


def _sc_body_counts(table_p, table_a, src2d, dst2d, zeros, ones, sums, cnts,
                    acc, sbuf0, sbuf1, sbuf2, sbuf3,
                    dbuf0, dbuf1, dbuf2, dbuf3, rows0, rows1,
                    isem0, isem1, isem2, isem3, gsem0, gsem1):
    _sc_body(table_p, table_a, src2d, dst2d, zeros, ones, sums, cnts, acc,
             sbuf0, sbuf1, sbuf2, sbuf3, dbuf0, dbuf1, dbuf2, dbuf3,
             rows0, rows1, isem0, isem1, isem2, isem3, gsem0, gsem1)


def _sc_body_sums(table_p, table_a, src2d, dst2d, zeros, sums, acc,
                  sbuf0, sbuf1, sbuf2, sbuf3,
                  dbuf0, dbuf1, dbuf2, dbuf3, rows0, rows1,
                  isem0, isem1, isem2, isem3, gsem0, gsem1):
    _sc_body(table_p, table_a, src2d, dst2d, zeros, None, sums, None, acc,
             sbuf0, sbuf1, sbuf2, sbuf3, dbuf0, dbuf1, dbuf2, dbuf3,
             rows0, rows1, isem0, isem1, isem2, isem3, gsem0, gsem1)


def _sc_scratch():
    return ([pltpu.VMEM_SHARED((NA, D), _f32)]
            + [pltpu.VMEM((K,), jnp.int32) for _ in range(8)]
            + [pltpu.VMEM((K, D), _f32) for _ in range(2)]
            + [pltpu.SemaphoreType.DMA for _ in range(6)])


@functools.cache
def _sc_agg_counts():
    mesh = plsc.VectorSubcoreMesh(core_axis_name="c", subcore_axis_name="s",
                                  num_cores=NC, num_subcores=NS)
    return pl.kernel(
        _sc_body_counts,
        out_type=[jax.ShapeDtypeStruct((NC, NA, D), _f32),
                  jax.ShapeDtypeStruct((NC, NA, D), _f32)],
        mesh=mesh,
        scratch_types=_sc_scratch(),
    )


@functools.cache
def _sc_agg():
    mesh = plsc.VectorSubcoreMesh(core_axis_name="c", subcore_axis_name="s",
                                  num_cores=NC, num_subcores=NS)
    return pl.kernel(
        _sc_body_sums,
        out_type=[jax.ShapeDtypeStruct((NC, NA, D), _f32)],
        mesh=mesh,
        scratch_types=_sc_scratch(),
    )


BR = 1000


def _dot(a, b):
    return lax.dot_general(a, b, (((1,), (0,)), ((), ())),
                           preferred_element_type=_f32)


def _tc0_body(x_ref, emb_ref, sums_ref, cnts_ref, wrp_ref, brp_ref,
              wra_ref, bra_ref, rc_ref, rw_ref, h1_ref, a1_ref, h1t_ref):
    x = x_ref[...]
    cnt_c = jnp.maximum(cnts_ref[0, :, 0:1], 1.0)
    cnt_w = jnp.maximum(cnts_ref[1, :, 0:1], 1.0)
    agg_c = sums_ref[0] / cnt_c
    agg_w = sums_ref[1] / cnt_w
    wrp = wrp_ref[...]
    rc = rc_ref[...]
    rw = rw_ref[...]
    brp = brp_ref[...]
    out = _dot(x, wrp) + brp + _dot(agg_c, rc) + _dot(agg_w, rw)
    h1_ref[...] = jnp.maximum(out, 0.0)
    a1_ref[...] = jnp.maximum(_dot(emb_ref[...], wra_ref[...]) + bra_ref[...], 0.0)
    h1t_ref[...] = jnp.maximum(_dot(x, wrp + rc + rw) + brp, 0.0)


def _tc1_body(h1_ref, h1t_ref, sums_ref, cnts_ref, wrp_ref, brp_ref,
              rc_ref, rw_ref, wout_ref, bout_ref, logits_ref, alpha_ref):
    h1 = h1_ref[...]
    h1t = h1t_ref[...]
    cnt_c = jnp.maximum(cnts_ref[0, :, 0:1], 1.0)
    cnt_w = jnp.maximum(cnts_ref[1, :, 0:1], 1.0)
    agg_c = sums_ref[0] / cnt_c
    agg_w = sums_ref[1] / cnt_w
    wrp = wrp_ref[...]
    rc = rc_ref[...]
    rw = rw_ref[...]
    brp = brp_ref[...]
    h2 = jnp.maximum(_dot(h1, wrp) + brp + _dot(agg_c, rc) + _dot(agg_w, rw), 0.0)
    h2t = jnp.maximum(_dot(h1t, wrp + rc + rw) + brp, 0.0)
    num0 = jnp.sum(h1 * h1t, axis=-1, keepdims=True)
    den0 = (jnp.sqrt(jnp.sum(h1 * h1, axis=-1, keepdims=True))
            * jnp.sqrt(jnp.sum(h1t * h1t, axis=-1, keepdims=True)) + 1e-8)
    s0 = num0 / den0
    num1 = jnp.sum(h2 * h2t, axis=-1, keepdims=True)
    den1 = (jnp.sqrt(jnp.sum(h2 * h2, axis=-1, keepdims=True))
            * jnp.sqrt(jnp.sum(h2t * h2t, axis=-1, keepdims=True)) + 1e-8)
    s1 = num1 / den1
    m = jnp.maximum(s0, s1)
    e0 = jnp.exp(s0 - m)
    e1 = jnp.exp(s1 - m)
    tot = e0 + e1
    a0 = e0 / tot
    a1 = e1 / tot
    h = a0 * h1 + a1 * h2
    logits_ref[...] = _dot(h, wout_ref[...]) + bout_ref[...]
    lane = lax.broadcasted_iota(jnp.int32, (BR, D), 1)
    alpha_ref[...] = jnp.where(lane == 0, a0, jnp.where(lane == 1, a1, 0.0))


def _row_spec(shape):
    nd = len(shape)
    if nd == 2:
        return pl.BlockSpec((BR, shape[1]), lambda i: (i, 0))
    return pl.BlockSpec((shape[0], BR, shape[2]), lambda i: (0, i, 0))


def _full_spec(shape):
    nd = len(shape)
    return pl.BlockSpec(shape, lambda i: (0,) * nd)


def _tc_layer0(x, emb, sums, cnts, rc, rw, wrp, brp, wra, bra):
    grid = (N // BR,)
    return pl.pallas_call(
        _tc0_body,
        grid=grid,
        in_specs=[
            _row_spec((N, D)), _row_spec((N, D)),
            _row_spec((NC, N, D)), _row_spec((NC, N, D)),
            _full_spec((D, D)), _full_spec((1, D)),
            _full_spec((D, D)), _full_spec((1, D)),
            _full_spec((D, D)), _full_spec((D, D)),
        ],
        out_specs=[_row_spec((N, D)), _row_spec((N, D)), _row_spec((N, D))],
        out_shape=[jax.ShapeDtypeStruct((N, D), _f32)] * 3,
    )(x, emb, sums, cnts, wrp, brp.reshape(1, D), wra, bra.reshape(1, D), rc, rw)


def _tc_layer1(h1, h1t, sums, cnts, rc, rw, wrp, brp, wout_p, bout_p):
    grid = (N // BR,)
    return pl.pallas_call(
        _tc1_body,
        grid=grid,
        in_specs=[
            _row_spec((N, D)), _row_spec((N, D)),
            _row_spec((NC, N, D)), _row_spec((NC, N, D)),
            _full_spec((D, D)), _full_spec((1, D)),
            _full_spec((D, D)), _full_spec((D, D)),
            _full_spec((D, NCLS)), _full_spec((1, NCLS)),
        ],
        out_specs=[_row_spec((N, NCLS)), _row_spec((N, D))],
        out_shape=[jax.ShapeDtypeStruct((N, NCLS), _f32),
                   jax.ShapeDtypeStruct((N, D), _f32)],
    )(h1, h1t, sums, cnts, wrp, brp.reshape(1, D), rc, rw, wout_p, bout_p)


def kernel(x_paper, emb_author, edge_cites, edge_writes,
           w_rel_cites_0, w_rel_writes_0, w_root_paper_0, b_root_paper_0,
           w_root_author_0, b_root_author_0,
           w_rel_cites_1, w_rel_writes_1, w_root_paper_1, b_root_paper_1,
           w_root_author_1, b_root_author_1, w_out, b_out):
    i32 = jnp.int32
    pad_src = jnp.zeros((PAD,), i32)
    pad_dst = jnp.full((PAD,), N, i32)
    src2d = jnp.concatenate(
        [edge_cites[0], pad_src, edge_writes[0], pad_src]).reshape(-1, K)
    dst2d = jnp.concatenate(
        [edge_cites[1], pad_dst, edge_writes[1], pad_dst]).reshape(-1, K)
    zeros128 = jnp.zeros((NA, D), _f32)
    ones = jnp.ones((K, D), _f32)

    sums0, cnts = _sc_agg_counts()(x_paper, emb_author, src2d, dst2d,
                                   zeros128, ones)
    h1, a1, h1t = _tc_layer0(x_paper, emb_author, sums0, cnts,
                             w_rel_cites_0, w_rel_writes_0,
                             w_root_paper_0, b_root_paper_0,
                             w_root_author_0, b_root_author_0)
    (sums1,) = _sc_agg()(h1, a1, src2d, dst2d, zeros128)
    logits, alpha_p = _tc_layer1(h1, h1t, sums1, cnts,
                                 w_rel_cites_1, w_rel_writes_1,
                                 w_root_paper_1, b_root_paper_1,
                                 w_out, b_out.reshape(1, NCLS))
    return logits, alpha_p[:, :2]
